# baseline (speedup 1.0000x reference)
import functools

import jax
import jax.numpy as jnp
from jax import lax
from jax.experimental import pallas as pl
from jax.experimental.pallas import tpu as pltpu
from jax.experimental.pallas import tpu_sc as plsc

_L = 16


def _build_sc_call(R, C, K, NC, NS):
    NW = NC * NS
    rows_per_w = R // NW
    CH = 32
    n_chunks = rows_per_w // CH
    n_groups = K // _L

    mesh = plsc.VectorSubcoreMesh(core_axis_name="c", subcore_axis_name="s")

    @functools.partial(
        pl.kernel,
        mesh=mesh,
        out_type=jax.ShapeDtypeStruct((R * K,), jnp.float32),
        scratch_types=[
            pltpu.VMEM((K,), jnp.int32),
            pltpu.VMEM((CH * C,), jnp.float32),
            pltpu.VMEM((CH * K,), jnp.float32),
        ],
        compiler_params=pltpu.CompilerParams(needs_layout_passes=False),
    )
    def sc_gather(x_hbm, idx_hbm, out_hbm, idx_v, x_v, out_v):
        cid = lax.axis_index("c")
        sid = lax.axis_index("s")
        wid = sid * NC + cid
        base = wid * rows_per_w

        pltpu.sync_copy(idx_hbm, idx_v)
        idx_vecs = [idx_v[pl.ds(g * _L, _L)] for g in range(n_groups)]

        def chunk_body(i, carry):
            r0 = base + i * CH
            pltpu.sync_copy(x_hbm.at[pl.ds(r0 * C, CH * C)], x_v)

            def row_body(r, c2):
                rowoff = r * C
                outoff = r * K
                for g in range(n_groups):
                    v = plsc.load_gather(x_v, [rowoff + idx_vecs[g]])
                    out_v[pl.ds(outoff + g * _L, _L)] = v
                return c2

            lax.fori_loop(0, CH, row_body, 0)
            pltpu.sync_copy(out_v, out_hbm.at[pl.ds(r0 * K, CH * K)])
            return carry

        lax.fori_loop(0, n_chunks, chunk_body, 0)

    return sc_gather


def kernel(X, feature_indices):
    R, C = X.shape
    K = feature_indices.shape[0]
    info = plsc.get_sparse_core_info()
    NC, NS = info.num_cores, info.num_subcores
    call = _build_sc_call(R, C, K, NC, NS)
    out_flat = call(X.reshape(-1), feature_indices.astype(jnp.int32))
    return out_flat.reshape(R, K)


# speedup vs baseline: 1.4041x; 1.4041x over previous
import functools

import jax
import jax.numpy as jnp
from jax import lax
from jax.experimental import pallas as pl
from jax.experimental.pallas import tpu as pltpu
from jax.experimental.pallas import tpu_sc as plsc

_L = 16


def _build_sc_call(R, C, K, NC, NS):
    NW = NC * NS
    rows_per_w = R // NW
    CH = 64
    n_chunks = rows_per_w // CH
    n_groups = K // _L

    mesh = plsc.VectorSubcoreMesh(core_axis_name="c", subcore_axis_name="s")

    @functools.partial(
        pl.kernel,
        mesh=mesh,
        out_type=jax.ShapeDtypeStruct((R * K,), jnp.float32),
        scratch_types=[
            pltpu.VMEM((K,), jnp.int32),
            pltpu.VMEM((CH * C,), jnp.float32),
            pltpu.VMEM((CH * C,), jnp.float32),
            pltpu.VMEM((CH * K,), jnp.float32),
            pltpu.VMEM((CH * K,), jnp.float32),
            pltpu.SemaphoreType.DMA,
            pltpu.SemaphoreType.DMA,
        ],
        compiler_params=pltpu.CompilerParams(needs_layout_passes=False),
    )
    def sc_gather(x_hbm, idx_hbm, out_hbm, idx_v, x0, x1, o0, o1,
                  sem_in, sem_out):
        cid = lax.axis_index("c")
        sid = lax.axis_index("s")
        wid = sid * NC + cid
        base = wid * rows_per_w
        xb = (x0, x1)
        ob = (o0, o1)

        pltpu.sync_copy(idx_hbm, idx_v)
        idx_vecs = [idx_v[pl.ds(g * _L, _L)] for g in range(n_groups)]

        def in_cp(i, b):
            r0 = base + i * CH
            return pltpu.make_async_copy(
                x_hbm.at[pl.ds(r0 * C, CH * C)], xb[b], sem_in)

        def out_cp(i, b):
            r0 = base + i * CH
            return pltpu.make_async_copy(
                ob[b], out_hbm.at[pl.ds(r0 * K, CH * K)], sem_out)

        def compute(x_v, out_v):
            @plsc.parallel_loop(0, CH, unroll=2)
            def row_body(r):
                rowvec = jnp.full((_L,), r * C, jnp.int32)
                outoff = r * K
                for g in range(n_groups):
                    v = plsc.load_gather(x_v, [rowvec + idx_vecs[g]])
                    out_v[pl.ds(outoff + g * _L, _L)] = v

        in_cp(0, 0).start()

        def ring_body(i0, carry):
            for b in range(2):
                i = i0 * 2 + b

                @pl.when(i + 1 < n_chunks)
                def _start_next():
                    in_cp(i + 1, 1 - b).start()

                in_cp(i, b).wait()

                @pl.when(i >= 2)
                def _free_out():
                    out_cp(i - 2, b).wait()

                compute(xb[b], ob[b])
                out_cp(i, b).start()
            return carry

        lax.fori_loop(0, n_chunks // 2, ring_body, 0)
        out_cp(n_chunks - 2, 0).wait()
        out_cp(n_chunks - 1, 1).wait()

    return sc_gather


def kernel(X, feature_indices):
    R, C = X.shape
    K = feature_indices.shape[0]
    info = plsc.get_sparse_core_info()
    NC, NS = info.num_cores, info.num_subcores
    call = _build_sc_call(R, C, K, NC, NS)
    out_flat = call(X.reshape(-1), feature_indices.astype(jnp.int32))
    return out_flat.reshape(R, K)
